# baseline (speedup 1.0000x reference)
import jax
import jax.numpy as jnp
import numpy as np
from jax.experimental import pallas as pl
from jax.experimental.pallas import tpu as pltpu
from jax.experimental.pallas import tpu_sc as plsc

K = 128
B = 64
N = 8192
SEG_VREGS = 8
SEGS = N // (SEG_VREGS * 16)
CAP = 512
TRIG = CAP - SEG_VREGS * 16
BUF = CAP + 64
NEG = np.float32(-np.inf)


def _vsort(v, desc):
    s, _ = plsc.sort_key_val(v, v, descending=desc)
    return s


def _rev(v):
    return jax.lax.rev(v, (0,))


def _ce(vs, i, j, desc):
    a, b = vs[i], vs[j]
    if desc:
        vs[i], vs[j] = jnp.maximum(a, b), jnp.minimum(a, b)
    else:
        vs[i], vs[j] = jnp.minimum(a, b), jnp.maximum(a, b)


def _merge_blocks(vs, desc):
    vs = list(vs)
    n = len(vs)
    s = n // 2
    while s >= 1:
        for base in range(0, n, 2 * s):
            for i in range(base, base + s):
                _ce(vs, i, i + s, desc)
        s //= 2
    return [_vsort(v, desc) for v in vs]


def _sort128(vs):
    r = [_vsort(vs[i], desc=(i % 2 == 0)) for i in range(8)]
    for p in range(4):
        r[2 * p:2 * p + 2] = _merge_blocks(r[2 * p:2 * p + 2], desc=(p % 2 == 0))
    for q in range(2):
        r[4 * q:4 * q + 4] = _merge_blocks(r[4 * q:4 * q + 4], desc=(q % 2 == 0))
    return _merge_blocks(r, desc=True)


def _merge_top(a, b):
    c = [jnp.maximum(a[j], _rev(b[7 - j])) for j in range(8)]
    return _merge_blocks(c, desc=True)


def _load_block(ref, base, cnt, iota):
    vs = []
    for j in range(8):
        off = base + j * 16
        v = ref[pl.ds(off, 16)]
        m = (off + iota) < cnt
        vs.append(jnp.where(m, v, NEG))
    return vs


def _select_top(cbuf, cnt, iota):
    blocks = [_sort128(_load_block(cbuf, blk * 128, cnt, iota))
              for blk in range(CAP // 128)]
    r = blocks[0]
    for blk in blocks[1:]:
        r = _merge_top(r, blk)
    return r


def _popcnt(mask):
    pc = plsc.all_reduce_population_count(mask)
    return jax.lax.squeeze(jax.lax.slice(pc, (0,), (1,)), (0,))


def _sc_body(x_hbm, out_hbm, rows_v, cb, cd, outv, cref, tref, *sems):
    wid = jax.lax.axis_index("c") * 16 + jax.lax.axis_index("s")
    iota = jax.lax.iota(jnp.int32, 16)
    bufs = (cb, cd)

    cops = [pltpu.async_copy(x_hbm.at[wid * 2 + i],
                             rows_v.at[pl.ds(i * 2 * N, 2 * N)], sems[i])
            for i in range(2)]

    def overflow(ch):
        cbuf = bufs[ch]

        def run():
            r = _select_top(cbuf, cref[ch], iota)
            for j in range(8):
                cbuf[pl.ds(j * 16, 16)] = r[j]
            tref[ch] = jnp.min(r[7])
            cref[ch] = 128

        return run

    for i in range(2):
        row = wid * 2 + i
        cops[i].wait()
        row_base = i * 2 * N
        cref[0] = 0
        cref[1] = 0
        tref[0] = NEG
        tref[1] = NEG

        def seg_body(si, _c):
            pl.when((cref[0] >= TRIG) | (si >= SEGS))(overflow(0))
            pl.when((cref[1] >= TRIG) | (si >= SEGS))(overflow(1))

            @pl.when(si < SEGS)
            def _append():
                tb = tref[0]
                td = tref[1]
                sb = si * (SEG_VREGS * 16)

                @plsc.parallel_loop(0, SEG_VREGS, unroll=SEG_VREGS,
                                    carry=(cref[0], cref[1]))
                def final_cnt(j, c):
                    nb, nd = c
                    nidx = row_base + 2 * (sb + j * 16 + iota)
                    vb = plsc.load_gather(rows_v, [nidx])
                    vd = plsc.load_gather(rows_v, [nidx + 1])
                    mb = vb > tb
                    md = vd > td
                    plsc.store_compressed(cb.at[pl.ds(nb, 16)], vb, mask=mb)
                    plsc.store_compressed(cd.at[pl.ds(nd, 16)], vd, mask=md)
                    return (nb + _popcnt(mb), nd + _popcnt(md))

                cref[0], cref[1] = final_cnt

            return _c

        jax.lax.fori_loop(0, SEGS + 1, seg_body, 0)

        for ch in range(2):
            cbuf = bufs[ch]
            for j in range(8):
                outv[pl.ds(j * 16, 16)] = cbuf[pl.ds(j * 16, 16)]
            pltpu.sync_copy(outv, out_hbm.at[row, pl.ds(ch * K, K)])


def kernel(diagrams):
    x2d = diagrams.reshape(B, 2 * N)
    mesh = plsc.VectorSubcoreMesh(core_axis_name="c", subcore_axis_name="s")
    k = pl.kernel(
        _sc_body,
        mesh=mesh,
        out_type=jax.ShapeDtypeStruct((B, 2 * K), jnp.float32),
        compiler_params=pltpu.CompilerParams(needs_layout_passes=False),
        scratch_types=[
            pltpu.VMEM((4 * N,), jnp.float32),
            pltpu.VMEM((BUF,), jnp.float32),
            pltpu.VMEM((BUF,), jnp.float32),
            pltpu.VMEM((K,), jnp.float32),
            pltpu.SMEM((2,), jnp.int32),
            pltpu.SMEM((2,), jnp.float32),
            pltpu.SemaphoreType.DMA,
            pltpu.SemaphoreType.DMA,
        ],
    )
    return k(x2d)


# speedup vs baseline: 1.2795x; 1.2795x over previous
import jax
import jax.numpy as jnp
import numpy as np
from jax.experimental import pallas as pl
from jax.experimental.pallas import tpu as pltpu
from jax.experimental.pallas import tpu_sc as plsc

K = 128
B = 64
N = 8192
SEG_VREGS = 8
SEGS = N // (SEG_VREGS * 16)
CAP = 512
TRIG = CAP - SEG_VREGS * 16
BUF = CAP + 64
NEG = np.float32(-np.inf)


def _vsort(v, desc):
    s, _ = plsc.sort_key_val(v, v, descending=desc)
    return s


def _rev(v):
    return jax.lax.rev(v, (0,))


def _ce(vs, i, j, desc):
    a, b = vs[i], vs[j]
    if desc:
        vs[i], vs[j] = jnp.maximum(a, b), jnp.minimum(a, b)
    else:
        vs[i], vs[j] = jnp.minimum(a, b), jnp.maximum(a, b)


def _merge_blocks(vs, desc):
    vs = list(vs)
    n = len(vs)
    s = n // 2
    while s >= 1:
        for base in range(0, n, 2 * s):
            for i in range(base, base + s):
                _ce(vs, i, i + s, desc)
        s //= 2
    return [_vsort(v, desc) for v in vs]


def _sort128(vs):
    r = [_vsort(vs[i], desc=(i % 2 == 0)) for i in range(8)]
    for p in range(4):
        r[2 * p:2 * p + 2] = _merge_blocks(r[2 * p:2 * p + 2], desc=(p % 2 == 0))
    for q in range(2):
        r[4 * q:4 * q + 4] = _merge_blocks(r[4 * q:4 * q + 4], desc=(q % 2 == 0))
    return _merge_blocks(r, desc=True)


def _merge_top(a, b):
    c = [jnp.maximum(a[j], _rev(b[7 - j])) for j in range(8)]
    return _merge_blocks(c, desc=True)


def _load_block(ref, base, cnt, iota):
    vs = []
    for j in range(8):
        off = base + j * 16
        v = ref[pl.ds(off, 16)]
        m = (off + iota) < cnt
        vs.append(jnp.where(m, v, NEG))
    return vs


def _select_top(cbuf, cnt, iota):
    blocks = [_sort128(_load_block(cbuf, blk * 128, cnt, iota))
              for blk in range(CAP // 128)]
    r = blocks[0]
    for blk in blocks[1:]:
        r = _merge_top(r, blk)
    return r


def _popcnt(mask):
    pc = plsc.all_reduce_population_count(mask)
    return jax.lax.squeeze(jax.lax.slice(pc, (0,), (1,)), (0,))


def _sc_body(x_hbm, out_hbm, rows_v, cb, cd, outv, cref, tref, *sems):
    wid = jax.lax.axis_index("c") * 16 + jax.lax.axis_index("s")
    iota = jax.lax.iota(jnp.int32, 16)
    bufs = (cb, cd)

    cops = [pltpu.async_copy(x_hbm.at[pl.ds((wid * 2 + i) * 2 * N, 2 * N)],
                             rows_v.at[pl.ds(i * 2 * N, 2 * N)], sems[i])
            for i in range(2)]

    def overflow(ch):
        cbuf = bufs[ch]

        def run():
            r = _select_top(cbuf, cref[ch], iota)
            for j in range(8):
                cbuf[pl.ds(j * 16, 16)] = r[j]
            tref[ch] = jnp.min(r[7])
            cref[ch] = 128

        return run

    for i in range(2):
        row = wid * 2 + i
        cops[i].wait()
        row_base = i * 2 * N
        cref[0] = 0
        cref[1] = 0
        tref[0] = NEG
        tref[1] = NEG

        def seg_body(si, _c):
            pl.when((cref[0] >= TRIG) | (si >= SEGS))(overflow(0))
            pl.when((cref[1] >= TRIG) | (si >= SEGS))(overflow(1))

            @pl.when(si < SEGS)
            def _append():
                tb = tref[0]
                td = tref[1]
                sb = si * 256

                @plsc.parallel_loop(0, SEG_VREGS, unroll=SEG_VREGS,
                                    carry=(cref[0], cref[1]))
                def final_cnt(j, c):
                    nb, nd = c
                    vb = rows_v[pl.ds(row_base + sb + j * 16, 16)]
                    vd = rows_v[pl.ds(row_base + sb + 128 + j * 16, 16)]
                    mb = vb > tb
                    md = vd > td
                    plsc.store_compressed(cb.at[pl.ds(nb, 16)], vb, mask=mb)
                    plsc.store_compressed(cd.at[pl.ds(nd, 16)], vd, mask=md)
                    return (nb + _popcnt(mb), nd + _popcnt(md))

                cref[0], cref[1] = final_cnt

            return _c

        jax.lax.fori_loop(0, SEGS + 1, seg_body, 0)

        obase = (row // 8) * 2048 + (row % 8) * 128
        for ch in range(2):
            cbuf = bufs[ch]
            for j in range(8):
                outv[pl.ds(j * 16, 16)] = cbuf[pl.ds(j * 16, 16)]
            pltpu.sync_copy(outv, out_hbm.at[pl.ds(obase + ch * 1024, K)])


def kernel(diagrams):
    x1d = diagrams.reshape(B, N // 128, 128, 2).transpose(0, 1, 3, 2).reshape(B * 2 * N)
    mesh = plsc.VectorSubcoreMesh(core_axis_name="c", subcore_axis_name="s")
    k = pl.kernel(
        _sc_body,
        mesh=mesh,
        out_type=jax.ShapeDtypeStruct((2 * K * B,), jnp.float32),
        compiler_params=pltpu.CompilerParams(needs_layout_passes=False),
        scratch_types=[
            pltpu.VMEM((4 * N,), jnp.float32),
            pltpu.VMEM((BUF,), jnp.float32),
            pltpu.VMEM((BUF,), jnp.float32),
            pltpu.VMEM((K,), jnp.float32),
            pltpu.SMEM((2,), jnp.int32),
            pltpu.SMEM((2,), jnp.float32),
            pltpu.SemaphoreType.DMA,
            pltpu.SemaphoreType.DMA,
        ],
    )
    out_flat = k(x1d)
    return out_flat.reshape(8, 2, 8, K).transpose(0, 2, 1, 3).reshape(B, 2 * K)


# speedup vs baseline: 2.3668x; 1.8498x over previous
import jax
import jax.numpy as jnp
import numpy as np
from jax.experimental import pallas as pl
from jax.experimental.pallas import tpu as pltpu
from jax.experimental.pallas import tpu_sc as plsc

K = 128
B = 64
N = 8192
SEG_VREGS = 8
SEGS = N // (SEG_VREGS * 16)
CAP = 512
TRIG = CAP - SEG_VREGS * 16
BUF = CAP + 64
NEG = np.float32(-np.inf)


def _vsort(v, desc):
    s, _ = plsc.sort_key_val(v, v, descending=desc)
    return s


def _rev(v):
    return jax.lax.rev(v, (0,))


def _ce(vs, i, j, desc):
    a, b = vs[i], vs[j]
    if desc:
        vs[i], vs[j] = jnp.maximum(a, b), jnp.minimum(a, b)
    else:
        vs[i], vs[j] = jnp.minimum(a, b), jnp.maximum(a, b)


def _merge_blocks(vs, desc):
    vs = list(vs)
    n = len(vs)
    s = n // 2
    while s >= 1:
        for base in range(0, n, 2 * s):
            for i in range(base, base + s):
                _ce(vs, i, i + s, desc)
        s //= 2
    return [_vsort(v, desc) for v in vs]


def _sort128(vs):
    r = [_vsort(vs[i], desc=(i % 2 == 0)) for i in range(8)]
    for p in range(4):
        r[2 * p:2 * p + 2] = _merge_blocks(r[2 * p:2 * p + 2], desc=(p % 2 == 0))
    for q in range(2):
        r[4 * q:4 * q + 4] = _merge_blocks(r[4 * q:4 * q + 4], desc=(q % 2 == 0))
    return _merge_blocks(r, desc=True)


def _merge_top(a, b):
    c = [jnp.maximum(a[j], _rev(b[7 - j])) for j in range(8)]
    return _merge_blocks(c, desc=True)


def _load_block(ref, base, rel, cnt, iota):
    vs = []
    for j in range(8):
        off = rel + j * 16
        v = ref[pl.ds(base + off, 16)]
        m = (off + iota) < cnt
        vs.append(jnp.where(m, v, NEG))
    return vs


def _select_top(cbuf, base, cnt, iota):
    blocks = [_sort128(_load_block(cbuf, base, blk * 128, cnt, iota))
              for blk in range(CAP // 128)]
    r = blocks[0]
    for blk in blocks[1:]:
        r = _merge_top(r, blk)
    return r


def _popcnt(mask):
    pc = plsc.all_reduce_population_count(mask)
    return jax.lax.squeeze(jax.lax.slice(pc, (0,), (1,)), (0,))


def _sc_body(x_hbm, out_hbm, rows_v, cbufs, outv, cref, tref, *sems):
    wid = jax.lax.axis_index("c") * 16 + jax.lax.axis_index("s")
    iota = jax.lax.iota(jnp.int32, 16)

    sem = sems[0]
    for i in range(2):
        pltpu.async_copy(x_hbm.at[pl.ds((wid * 2 + i) * 2 * N, 2 * N)],
                         rows_v.at[pl.ds(i * 2 * N, 2 * N)], sem.at[i])

    def row_body(i, _r):
        row = wid * 2 + i
        row_base = i * 2 * N
        pltpu.make_async_copy(
            x_hbm.at[pl.ds((wid * 2 + i) * 2 * N, 2 * N)],
            rows_v.at[pl.ds(row_base, 2 * N)], sem.at[i]).wait()
        cref[0] = 0
        cref[1] = 0
        tref[0] = NEG
        tref[1] = NEG

        def seg_body(si, _c):
            def chk(ch, _k):
                chbase = ch * BUF

                @pl.when((cref[ch] >= TRIG) | (si >= SEGS))
                def _of():
                    r = _select_top(cbufs, chbase, cref[ch], iota)
                    for j in range(8):
                        cbufs[pl.ds(chbase + j * 16, 16)] = r[j]
                    tref[ch] = jnp.min(r[7])
                    cref[ch] = 128

                return _k

            jax.lax.fori_loop(0, 2, chk, 0)

            @pl.when(si < SEGS)
            def _append():
                tb = tref[0]
                td = tref[1]
                sb = si * 256

                @plsc.parallel_loop(0, SEG_VREGS, unroll=SEG_VREGS,
                                    carry=(cref[0], cref[1]))
                def final_cnt(j, c):
                    nb, nd = c
                    vb = rows_v[pl.ds(row_base + sb + j * 16, 16)]
                    vd = rows_v[pl.ds(row_base + sb + 128 + j * 16, 16)]
                    mb = vb > tb
                    md = vd > td
                    plsc.store_compressed(cbufs.at[pl.ds(nb, 16)], vb, mask=mb)
                    plsc.store_compressed(cbufs.at[pl.ds(BUF + nd, 16)], vd, mask=md)
                    return (nb + _popcnt(mb), nd + _popcnt(md))

                cref[0], cref[1] = final_cnt

            return _c

        jax.lax.fori_loop(0, SEGS + 1, seg_body, 0)

        obase = (row // 8) * 2048 + (row % 8) * 128

        def out_ch(ch, _k):
            for j in range(8):
                outv[pl.ds(j * 16, 16)] = cbufs[pl.ds(ch * BUF + j * 16, 16)]
            pltpu.sync_copy(outv, out_hbm.at[pl.ds(obase + ch * 1024, K)])
            return _k

        jax.lax.fori_loop(0, 2, out_ch, 0)
        return _r

    jax.lax.fori_loop(0, 2, row_body, 0)


def kernel(diagrams):
    x1d = diagrams.reshape(B, N // 128, 128, 2).transpose(0, 1, 3, 2).reshape(B * 2 * N)
    mesh = plsc.VectorSubcoreMesh(core_axis_name="c", subcore_axis_name="s")
    k = pl.kernel(
        _sc_body,
        mesh=mesh,
        out_type=jax.ShapeDtypeStruct((2 * K * B,), jnp.float32),
        compiler_params=pltpu.CompilerParams(needs_layout_passes=False),
        scratch_types=[
            pltpu.VMEM((4 * N,), jnp.float32),
            pltpu.VMEM((2 * BUF,), jnp.float32),
            pltpu.VMEM((K,), jnp.float32),
            pltpu.SMEM((2,), jnp.int32),
            pltpu.SMEM((2,), jnp.float32),
            pltpu.SemaphoreType.DMA((2,)),
        ],
    )
    out_flat = k(x1d)
    return out_flat.reshape(8, 2, 8, K).transpose(0, 2, 1, 3).reshape(B, 2 * K)


# speedup vs baseline: 2.4764x; 1.0463x over previous
import jax
import jax.numpy as jnp
import numpy as np
from jax.experimental import pallas as pl
from jax.experimental.pallas import tpu as pltpu
from jax.experimental.pallas import tpu_sc as plsc

K = 128
B = 64
N = 8192
SEG_VREGS = 8
SEGS = N // (SEG_VREGS * 16)
CAP = 512
TRIG = CAP - SEG_VREGS * 16
BUF = CAP + 64
NEG = np.float32(-np.inf)


def _vsort(v, desc):
    s, _ = plsc.sort_key_val(v, v, descending=desc)
    return s


def _rev(v):
    return jax.lax.rev(v, (0,))


def _ce(vs, i, j, desc):
    a, b = vs[i], vs[j]
    if desc:
        vs[i], vs[j] = jnp.maximum(a, b), jnp.minimum(a, b)
    else:
        vs[i], vs[j] = jnp.minimum(a, b), jnp.maximum(a, b)


def _merge_blocks(vs, desc):
    vs = list(vs)
    n = len(vs)
    s = n // 2
    while s >= 1:
        for base in range(0, n, 2 * s):
            for i in range(base, base + s):
                _ce(vs, i, i + s, desc)
        s //= 2
    return [_vsort(v, desc) for v in vs]


def _sort128(vs):
    r = [_vsort(vs[i], desc=(i % 2 == 0)) for i in range(8)]
    for p in range(4):
        r[2 * p:2 * p + 2] = _merge_blocks(r[2 * p:2 * p + 2], desc=(p % 2 == 0))
    for q in range(2):
        r[4 * q:4 * q + 4] = _merge_blocks(r[4 * q:4 * q + 4], desc=(q % 2 == 0))
    return _merge_blocks(r, desc=True)


def _merge_top(a, b):
    c = [jnp.maximum(a[j], _rev(b[7 - j])) for j in range(8)]
    return _merge_blocks(c, desc=True)


def _load_block(ref, base, rel, cnt, iota):
    vs = []
    for j in range(8):
        off = rel + j * 16
        v = ref[pl.ds(base + off, 16)]
        m = (off + iota) < cnt
        vs.append(jnp.where(m, v, NEG))
    return vs


def _select_top(cbuf, base, cnt, iota):
    def sort_blk(blk, _k):
        bb = base + blk * 128
        r = _sort128(_load_block(cbuf, base, blk * 128, cnt, iota))
        for j in range(8):
            cbuf[pl.ds(bb + j * 16, 16)] = r[j]
        return _k

    jax.lax.fori_loop(0, CAP // 128, sort_blk, 0)

    def merge_blk(blk, r):
        b = [cbuf[pl.ds(base + blk * 128 + j * 16, 16)] for j in range(8)]
        return tuple(_merge_top(list(r), b))

    r0 = tuple(cbuf[pl.ds(base + j * 16, 16)] for j in range(8))
    return list(jax.lax.fori_loop(1, CAP // 128, merge_blk, r0))


def _popcnt(mask):
    pc = plsc.all_reduce_population_count(mask)
    return jax.lax.squeeze(jax.lax.slice(pc, (0,), (1,)), (0,))


def _sc_body(x_hbm, out_hbm, rows_v, cbufs, outv, cref, tref, *sems):
    wid = jax.lax.axis_index("c") * 16 + jax.lax.axis_index("s")
    iota = jax.lax.iota(jnp.int32, 16)

    sem = sems[0]
    for i in range(2):
        pltpu.async_copy(x_hbm.at[pl.ds((wid * 2 + i) * 2 * N, 2 * N)],
                         rows_v.at[pl.ds(i * 2 * N, 2 * N)], sem.at[i])

    def row_body(i, _r):
        row = wid * 2 + i
        row_base = i * 2 * N
        pltpu.make_async_copy(
            x_hbm.at[pl.ds((wid * 2 + i) * 2 * N, 2 * N)],
            rows_v.at[pl.ds(row_base, 2 * N)], sem.at[i]).wait()
        cref[0] = 0
        cref[1] = 0
        tref[0] = NEG
        tref[1] = NEG

        def seg_body(si, _c):
            @pl.when((cref[0] >= TRIG) | (cref[1] >= TRIG) | (si >= SEGS))
            def _sel():
                def chk(ch, _k):
                    chbase = ch * BUF

                    @pl.when((cref[ch] >= TRIG) | (si >= SEGS))
                    def _of():
                        r = _select_top(cbufs, chbase, cref[ch], iota)
                        for j in range(8):
                            cbufs[pl.ds(chbase + j * 16, 16)] = r[j]
                        tref[ch] = jnp.min(r[7])
                        cref[ch] = 128

                    return _k

                jax.lax.fori_loop(0, 2, chk, 0)

            @pl.when(si < SEGS)
            def _append():
                tb = tref[0]
                td = tref[1]
                sb = si * 256

                @plsc.parallel_loop(0, SEG_VREGS, unroll=SEG_VREGS,
                                    carry=(cref[0], cref[1]))
                def final_cnt(j, c):
                    nb, nd = c
                    vb = rows_v[pl.ds(row_base + sb + j * 16, 16)]
                    vd = rows_v[pl.ds(row_base + sb + 128 + j * 16, 16)]
                    mb = vb > tb
                    md = vd > td
                    plsc.store_compressed(cbufs.at[pl.ds(nb, 16)], vb, mask=mb)
                    plsc.store_compressed(cbufs.at[pl.ds(BUF + nd, 16)], vd, mask=md)
                    return (nb + _popcnt(mb), nd + _popcnt(md))

                cref[0], cref[1] = final_cnt

            return _c

        jax.lax.fori_loop(0, SEGS + 1, seg_body, 0)

        obase = (row // 8) * 2048 + (row % 8) * 128

        def out_ch(ch, _k):
            for j in range(8):
                outv[pl.ds(j * 16, 16)] = cbufs[pl.ds(ch * BUF + j * 16, 16)]
            pltpu.sync_copy(outv, out_hbm.at[pl.ds(obase + ch * 1024, K)])
            return _k

        jax.lax.fori_loop(0, 2, out_ch, 0)
        return _r

    jax.lax.fori_loop(0, 2, row_body, 0)


def kernel(diagrams):
    x1d = diagrams.reshape(B, N // 128, 128, 2).transpose(0, 1, 3, 2).reshape(B * 2 * N)
    mesh = plsc.VectorSubcoreMesh(core_axis_name="c", subcore_axis_name="s")
    k = pl.kernel(
        _sc_body,
        mesh=mesh,
        out_type=jax.ShapeDtypeStruct((2 * K * B,), jnp.float32),
        compiler_params=pltpu.CompilerParams(needs_layout_passes=False),
        scratch_types=[
            pltpu.VMEM((4 * N,), jnp.float32),
            pltpu.VMEM((2 * BUF,), jnp.float32),
            pltpu.VMEM((K,), jnp.float32),
            pltpu.SMEM((2,), jnp.int32),
            pltpu.SMEM((2,), jnp.float32),
            pltpu.SemaphoreType.DMA((2,)),
        ],
    )
    out_flat = k(x1d)
    return out_flat.reshape(8, 2, 8, K).transpose(0, 2, 1, 3).reshape(B, 2 * K)


# speedup vs baseline: 2.4877x; 1.0046x over previous
import jax
import jax.numpy as jnp
import numpy as np
from jax.experimental import pallas as pl
from jax.experimental.pallas import tpu as pltpu
from jax.experimental.pallas import tpu_sc as plsc

K = 128
B = 64
N = 8192
SEG_VREGS = 8
SEGS = N // (SEG_VREGS * 16)
CAP = 512
TRIG = CAP - SEG_VREGS * 16
BUF = CAP + 64
NEG = np.float32(-np.inf)


def _vsort(v, desc):
    s, _ = plsc.sort_key_val(v, v, descending=desc)
    return s


def _rev(v):
    return jax.lax.rev(v, (0,))


def _ce(vs, i, j, desc):
    a, b = vs[i], vs[j]
    if desc:
        vs[i], vs[j] = jnp.maximum(a, b), jnp.minimum(a, b)
    else:
        vs[i], vs[j] = jnp.minimum(a, b), jnp.maximum(a, b)


def _merge_blocks(vs, desc):
    vs = list(vs)
    n = len(vs)
    s = n // 2
    while s >= 1:
        for base in range(0, n, 2 * s):
            for i in range(base, base + s):
                _ce(vs, i, i + s, desc)
        s //= 2
    return [_vsort(v, desc) for v in vs]


def _sort128(vs):
    r = [_vsort(vs[i], desc=(i % 2 == 0)) for i in range(8)]
    for p in range(4):
        r[2 * p:2 * p + 2] = _merge_blocks(r[2 * p:2 * p + 2], desc=(p % 2 == 0))
    for q in range(2):
        r[4 * q:4 * q + 4] = _merge_blocks(r[4 * q:4 * q + 4], desc=(q % 2 == 0))
    return _merge_blocks(r, desc=True)


def _merge_top(a, b):
    c = [jnp.maximum(a[j], _rev(b[7 - j])) for j in range(8)]
    return _merge_blocks(c, desc=True)


def _load_block(ref, base, rel, cnt, iota):
    vs = []
    for j in range(8):
        off = rel + j * 16
        v = ref[pl.ds(base + off, 16)]
        m = (off + iota) < cnt
        vs.append(jnp.where(m, v, NEG))
    return vs


def _select_top(cbuf, base, cnt, iota):
    def sort_blk(blk, _k):
        bb = base + blk * 128
        r = _sort128(_load_block(cbuf, base, blk * 128, cnt, iota))
        for j in range(8):
            cbuf[pl.ds(bb + j * 16, 16)] = r[j]
        return _k

    jax.lax.fori_loop(1, CAP // 128, sort_blk, 0)

    def merge_blk(blk, r):
        b = [cbuf[pl.ds(base + blk * 128 + j * 16, 16)] for j in range(8)]
        return tuple(_merge_top(list(r), b))

    r0 = tuple(cbuf[pl.ds(base + j * 16, 16)] for j in range(8))
    return list(jax.lax.fori_loop(1, CAP // 128, merge_blk, r0))


def _popcnt(mask):
    pc = plsc.all_reduce_population_count(mask)
    return jax.lax.squeeze(jax.lax.slice(pc, (0,), (1,)), (0,))


def _sc_body(x_hbm, out_hbm, rows_v, cbufs, outv, cref, tref, *sems):
    wid = jax.lax.axis_index("c") * 16 + jax.lax.axis_index("s")
    iota = jax.lax.iota(jnp.int32, 16)

    sem = sems[0]
    for i in range(2):
        pltpu.async_copy(x_hbm.at[pl.ds((wid * 2 + i) * 2 * N, 2 * N)],
                         rows_v.at[pl.ds(i * 2 * N, 2 * N)], sem.at[i])

    def row_body(i, _r):
        row = wid * 2 + i
        row_base = i * 2 * N
        pltpu.make_async_copy(
            x_hbm.at[pl.ds((wid * 2 + i) * 2 * N, 2 * N)],
            rows_v.at[pl.ds(row_base, 2 * N)], sem.at[i]).wait()
        neg16 = jnp.full((16,), NEG, jnp.float32)

        def seed(j, _k):
            cbufs[pl.ds(j * 16, 16)] = neg16
            cbufs[pl.ds(BUF + j * 16, 16)] = neg16
            return _k

        jax.lax.fori_loop(0, 8, seed, 0)
        cref[0] = 128
        cref[1] = 128
        tref[0] = NEG
        tref[1] = NEG

        def seg_body(si, _c):
            @pl.when((cref[0] >= TRIG) | (cref[1] >= TRIG) | (si >= SEGS))
            def _sel():
                def chk(ch, _k):
                    chbase = ch * BUF

                    @pl.when((cref[ch] >= TRIG) | (si >= SEGS))
                    def _of():
                        r = _select_top(cbufs, chbase, cref[ch], iota)
                        for j in range(8):
                            cbufs[pl.ds(chbase + j * 16, 16)] = r[j]
                        tref[ch] = jnp.min(r[7])
                        cref[ch] = 128

                    return _k

                jax.lax.fori_loop(0, 2, chk, 0)

            @pl.when(si < SEGS)
            def _append():
                tb = tref[0]
                td = tref[1]
                sb = si * 256

                @plsc.parallel_loop(0, SEG_VREGS, unroll=SEG_VREGS,
                                    carry=(cref[0], cref[1]))
                def final_cnt(j, c):
                    nb, nd = c
                    vb = rows_v[pl.ds(row_base + sb + j * 16, 16)]
                    vd = rows_v[pl.ds(row_base + sb + 128 + j * 16, 16)]
                    mb = vb > tb
                    md = vd > td
                    plsc.store_compressed(cbufs.at[pl.ds(nb, 16)], vb, mask=mb)
                    plsc.store_compressed(cbufs.at[pl.ds(BUF + nd, 16)], vd, mask=md)
                    return (nb + _popcnt(mb), nd + _popcnt(md))

                cref[0], cref[1] = final_cnt

            return _c

        jax.lax.fori_loop(0, SEGS + 1, seg_body, 0)

        obase = (row // 8) * 2048 + (row % 8) * 128

        def out_ch(ch, _k):
            for j in range(8):
                outv[pl.ds(j * 16, 16)] = cbufs[pl.ds(ch * BUF + j * 16, 16)]
            pltpu.sync_copy(outv, out_hbm.at[pl.ds(obase + ch * 1024, K)])
            return _k

        jax.lax.fori_loop(0, 2, out_ch, 0)
        return _r

    jax.lax.fori_loop(0, 2, row_body, 0)


def kernel(diagrams):
    x1d = diagrams.reshape(B, N // 128, 128, 2).transpose(0, 1, 3, 2).reshape(B * 2 * N)
    mesh = plsc.VectorSubcoreMesh(core_axis_name="c", subcore_axis_name="s")
    k = pl.kernel(
        _sc_body,
        mesh=mesh,
        out_type=jax.ShapeDtypeStruct((2 * K * B,), jnp.float32),
        compiler_params=pltpu.CompilerParams(needs_layout_passes=False),
        scratch_types=[
            pltpu.VMEM((4 * N,), jnp.float32),
            pltpu.VMEM((2 * BUF,), jnp.float32),
            pltpu.VMEM((K,), jnp.float32),
            pltpu.SMEM((2,), jnp.int32),
            pltpu.SMEM((2,), jnp.float32),
            pltpu.SemaphoreType.DMA((2,)),
        ],
    )
    out_flat = k(x1d)
    return out_flat.reshape(8, 2, 8, K).transpose(0, 2, 1, 3).reshape(B, 2 * K)


# speedup vs baseline: 2.4914x; 1.0015x over previous
import jax
import jax.numpy as jnp
import numpy as np
from jax.experimental import pallas as pl
from jax.experimental.pallas import tpu as pltpu
from jax.experimental.pallas import tpu_sc as plsc

K = 128
B = 64
N = 8192
SEG_VREGS = 8
SEGS = N // (SEG_VREGS * 16)
CAP = 512
TRIG = CAP - SEG_VREGS * 16
BUF = CAP + 64
NEG = np.float32(-np.inf)


def _vsort(v, desc):
    s, _ = plsc.sort_key_val(v, v, descending=desc)
    return s


def _rev(v):
    return jax.lax.rev(v, (0,))


def _ce(vs, i, j, desc):
    a, b = vs[i], vs[j]
    if desc:
        vs[i], vs[j] = jnp.maximum(a, b), jnp.minimum(a, b)
    else:
        vs[i], vs[j] = jnp.minimum(a, b), jnp.maximum(a, b)


def _merge_blocks(vs, desc):
    vs = list(vs)
    n = len(vs)
    s = n // 2
    while s >= 1:
        for base in range(0, n, 2 * s):
            for i in range(base, base + s):
                _ce(vs, i, i + s, desc)
        s //= 2
    return [_vsort(v, desc) for v in vs]


def _sort128(vs):
    r = [_vsort(vs[i], desc=(i % 2 == 0)) for i in range(8)]
    for p in range(4):
        r[2 * p:2 * p + 2] = _merge_blocks(r[2 * p:2 * p + 2], desc=(p % 2 == 0))
    for q in range(2):
        r[4 * q:4 * q + 4] = _merge_blocks(r[4 * q:4 * q + 4], desc=(q % 2 == 0))
    return _merge_blocks(r, desc=True)


def _merge_top(a, b):
    c = [jnp.maximum(a[j], _rev(b[7 - j])) for j in range(8)]
    return _merge_blocks(c, desc=True)


def _load_block(ref, base, rel, cnt, iota):
    vs = []
    for j in range(8):
        off = rel + j * 16
        v = ref[pl.ds(base + off, 16)]
        m = (off + iota) < cnt
        vs.append(jnp.where(m, v, NEG))
    return vs


def _select_top(cbuf, base, cnt, iota):
    def sort_blk(blk, _k):
        bb = base + blk * 128
        r = _sort128(_load_block(cbuf, base, blk * 128, cnt, iota))
        for j in range(8):
            cbuf[pl.ds(bb + j * 16, 16)] = r[j]
        return _k

    jax.lax.fori_loop(1, CAP // 128, sort_blk, 0)

    def merge_blk(blk, r):
        b = [cbuf[pl.ds(base + blk * 128 + j * 16, 16)] for j in range(8)]
        return tuple(_merge_top(list(r), b))

    r0 = tuple(cbuf[pl.ds(base + j * 16, 16)] for j in range(8))
    return list(jax.lax.fori_loop(1, CAP // 128, merge_blk, r0))


def _popcnt(mask):
    pc = plsc.all_reduce_population_count(mask)
    return jax.lax.squeeze(jax.lax.slice(pc, (0,), (1,)), (0,))


def _sc_body(x_hbm, out_hbm, rows_v, cbufs, cref, tref, *sems):
    wid = jax.lax.axis_index("c") * 16 + jax.lax.axis_index("s")
    iota = jax.lax.iota(jnp.int32, 16)

    sem = sems[0]
    for i in range(2):
        pltpu.async_copy(x_hbm.at[pl.ds((wid * 2 + i) * 2 * N, 2 * N)],
                         rows_v.at[pl.ds(i * 2 * N, 2 * N)], sem.at[i])

    def row_body(i, _r):
        row = wid * 2 + i
        row_base = i * 2 * N
        pltpu.make_async_copy(
            x_hbm.at[pl.ds((wid * 2 + i) * 2 * N, 2 * N)],
            rows_v.at[pl.ds(row_base, 2 * N)], sem.at[i]).wait()
        neg16 = jnp.full((16,), NEG, jnp.float32)

        def seed(j, _k):
            cbufs[pl.ds(j * 16, 16)] = neg16
            cbufs[pl.ds(BUF + j * 16, 16)] = neg16
            return _k

        jax.lax.fori_loop(0, 8, seed, 0)
        cref[0] = 128
        cref[1] = 128
        tref[0] = NEG
        tref[1] = NEG

        def seg_body(si, _c):
            @pl.when((cref[0] >= TRIG) | (cref[1] >= TRIG) | (si >= SEGS))
            def _sel():
                def chk(ch, _k):
                    chbase = ch * BUF

                    @pl.when((cref[ch] >= TRIG) | (si >= SEGS))
                    def _of():
                        r = _select_top(cbufs, chbase, cref[ch], iota)
                        for j in range(8):
                            cbufs[pl.ds(chbase + j * 16, 16)] = r[j]
                        tref[ch] = jnp.min(r[7])
                        cref[ch] = 128

                    return _k

                jax.lax.fori_loop(0, 2, chk, 0)

            @pl.when(si < SEGS)
            def _append():
                tb = tref[0]
                td = tref[1]
                sb = si * 256

                @plsc.parallel_loop(0, SEG_VREGS, unroll=SEG_VREGS,
                                    carry=(cref[0], cref[1]))
                def final_cnt(j, c):
                    nb, nd = c
                    vb = rows_v[pl.ds(row_base + sb + j * 16, 16)]
                    vd = rows_v[pl.ds(row_base + sb + 128 + j * 16, 16)]
                    mb = vb > tb
                    md = vd > td
                    plsc.store_compressed(cbufs.at[pl.ds(nb, 16)], vb, mask=mb)
                    plsc.store_compressed(cbufs.at[pl.ds(BUF + nd, 16)], vd, mask=md)
                    return (nb + _popcnt(mb), nd + _popcnt(md))

                cref[0], cref[1] = final_cnt

            return _c

        jax.lax.fori_loop(0, SEGS + 1, seg_body, 0)

        obase = (row // 8) * 2048 + (row % 8) * 128

        def out_ch(ch, _k):
            pltpu.sync_copy(cbufs.at[pl.ds(ch * BUF, K)],
                            out_hbm.at[pl.ds(obase + ch * 1024, K)])
            return _k

        jax.lax.fori_loop(0, 2, out_ch, 0)
        return _r

    jax.lax.fori_loop(0, 2, row_body, 0)


def kernel(diagrams):
    x1d = diagrams.reshape(B, N // 128, 128, 2).transpose(0, 1, 3, 2).reshape(B * 2 * N)
    mesh = plsc.VectorSubcoreMesh(core_axis_name="c", subcore_axis_name="s")
    k = pl.kernel(
        _sc_body,
        mesh=mesh,
        out_type=jax.ShapeDtypeStruct((2 * K * B,), jnp.float32),
        compiler_params=pltpu.CompilerParams(needs_layout_passes=False),
        scratch_types=[
            pltpu.VMEM((4 * N,), jnp.float32),
            pltpu.VMEM((2 * BUF,), jnp.float32),
            pltpu.SMEM((2,), jnp.int32),
            pltpu.SMEM((2,), jnp.float32),
            pltpu.SemaphoreType.DMA((2,)),
        ],
    )
    out_flat = k(x1d)
    return out_flat.reshape(8, 2, 8, K).transpose(0, 2, 1, 3).reshape(B, 2 * K)


# speedup vs baseline: 2.4919x; 1.0002x over previous
import jax
import jax.numpy as jnp
import numpy as np
from jax.experimental import pallas as pl
from jax.experimental.pallas import tpu as pltpu
from jax.experimental.pallas import tpu_sc as plsc

K = 128
B = 64
N = 8192
SEG_VREGS = 8
SEGS = N // (SEG_VREGS * 16)
CAP = 512
TRIG = CAP - SEG_VREGS * 16
BUF = CAP + 64
NEG = np.float32(-np.inf)


def _vsort(v, desc):
    s, _ = plsc.sort_key_val(v, v, descending=desc)
    return s


def _rev(v):
    return jax.lax.rev(v, (0,))


def _ce(vs, i, j, desc):
    a, b = vs[i], vs[j]
    if desc:
        vs[i], vs[j] = jnp.maximum(a, b), jnp.minimum(a, b)
    else:
        vs[i], vs[j] = jnp.minimum(a, b), jnp.maximum(a, b)


def _merge_blocks(vs, desc):
    vs = list(vs)
    n = len(vs)
    s = n // 2
    while s >= 1:
        for base in range(0, n, 2 * s):
            for i in range(base, base + s):
                _ce(vs, i, i + s, desc)
        s //= 2
    return [_vsort(v, desc) for v in vs]


def _sort128(vs):
    r = [_vsort(vs[i], desc=(i % 2 == 0)) for i in range(8)]
    for p in range(4):
        r[2 * p:2 * p + 2] = _merge_blocks(r[2 * p:2 * p + 2], desc=(p % 2 == 0))
    for q in range(2):
        r[4 * q:4 * q + 4] = _merge_blocks(r[4 * q:4 * q + 4], desc=(q % 2 == 0))
    return _merge_blocks(r, desc=True)


def _merge_top(a, b):
    c = [jnp.maximum(a[j], _rev(b[7 - j])) for j in range(8)]
    return _merge_blocks(c, desc=True)


def _load_block(ref, base, rel, cnt, iota):
    vs = []
    for j in range(8):
        off = rel + j * 16
        v = ref[pl.ds(base + off, 16)]
        m = (off + iota) < cnt
        vs.append(jnp.where(m, v, NEG))
    return vs


def _select_top(cbuf, base, cnt, iota):
    def sort_blk(blk, _k):
        bb = base + blk * 128
        r = _sort128(_load_block(cbuf, base, blk * 128, cnt, iota))
        for j in range(8):
            cbuf[pl.ds(bb + j * 16, 16)] = r[j]
        return _k

    jax.lax.fori_loop(1, CAP // 128, sort_blk, 0)

    def merge_blk(blk, r):
        b = [cbuf[pl.ds(base + blk * 128 + j * 16, 16)] for j in range(8)]
        return tuple(_merge_top(list(r), b))

    r0 = tuple(cbuf[pl.ds(base + j * 16, 16)] for j in range(8))
    return list(jax.lax.fori_loop(1, CAP // 128, merge_blk, r0))


def _popcnt(mask):
    pc = plsc.all_reduce_population_count(mask)
    return jax.lax.squeeze(jax.lax.slice(pc, (0,), (1,)), (0,))


def _sc_body(x_hbm, out_hbm, rows_v, cbufs, cref, tref, *sems):
    wid = jax.lax.axis_index("c") * 16 + jax.lax.axis_index("s")
    iota = jax.lax.iota(jnp.int32, 16)

    sem = sems[0]
    for i in range(2):
        pltpu.async_copy(x_hbm.at[pl.ds((wid * 2 + i) * 2 * N, 2 * N)],
                         rows_v.at[pl.ds(i * 2 * N, 2 * N)], sem.at[i])

    def row_body(i, _r):
        row = wid * 2 + i
        row_base = i * 2 * N
        pltpu.make_async_copy(
            x_hbm.at[pl.ds((wid * 2 + i) * 2 * N, 2 * N)],
            rows_v.at[pl.ds(row_base, 2 * N)], sem.at[i]).wait()
        neg16 = jnp.full((16,), NEG, jnp.float32)

        def seed(j, _k):
            cbufs[pl.ds(j * 16, 16)] = neg16
            cbufs[pl.ds(BUF + j * 16, 16)] = neg16
            return _k

        jax.lax.fori_loop(0, 8, seed, 0)
        cref[0] = 128
        cref[1] = 128
        tref[0] = NEG
        tref[1] = NEG

        def seg_body(si, _c):
            @pl.when((cref[0] >= TRIG) | (cref[1] >= TRIG) | (si >= SEGS))
            def _sel():
                def chk(ch, _k):
                    chbase = ch * BUF

                    @pl.when((cref[ch] >= TRIG) | (si >= SEGS))
                    def _of():
                        r = _select_top(cbufs, chbase, cref[ch], iota)
                        for j in range(8):
                            cbufs[pl.ds(chbase + j * 16, 16)] = r[j]
                        tref[ch] = jnp.min(r[7])
                        cref[ch] = 128

                    return _k

                jax.lax.fori_loop(0, 2, chk, 0)

            @pl.when(si < SEGS)
            def _append():
                tb = tref[0]
                td = tref[1]
                sb = si * 256

                @plsc.parallel_loop(0, SEG_VREGS, unroll=SEG_VREGS,
                                    carry=(cref[0], cref[1]))
                def final_cnt(j, c):
                    nb, nd = c
                    vb = rows_v[pl.ds(row_base + sb + j * 16, 16)]
                    vd = rows_v[pl.ds(row_base + sb + 128 + j * 16, 16)]
                    mb = vb > tb
                    md = vd > td
                    plsc.store_compressed(cbufs.at[pl.ds(nb, 16)], vb, mask=mb)
                    plsc.store_compressed(cbufs.at[pl.ds(BUF + nd, 16)], vd, mask=md)
                    return (nb + _popcnt(mb), nd + _popcnt(md))

                cref[0], cref[1] = final_cnt

            return _c

        jax.lax.fori_loop(0, SEGS + 1, seg_body, 0)

        obase = (row // 8) * 2048 + (row % 8) * 128

        def out_ch(ch, _k):
            pltpu.sync_copy(cbufs.at[pl.ds(ch * BUF, K)],
                            out_hbm.at[pl.ds(obase + ch * 1024, K)])
            return _k

        jax.lax.fori_loop(0, 2, out_ch, 0)
        return _r

    jax.lax.fori_loop(0, 2, row_body, 0)


def kernel(diagrams):
    x1d = diagrams.reshape(B, N // 128, 128, 2).transpose(0, 1, 3, 2).reshape(B * 2 * N)
    mesh = plsc.VectorSubcoreMesh(core_axis_name="c", subcore_axis_name="s")
    k = pl.kernel(
        _sc_body,
        mesh=mesh,
        out_type=jax.ShapeDtypeStruct((2 * K * B,), jnp.float32),
        compiler_params=pltpu.CompilerParams(
            needs_layout_passes=False, skip_device_barrier=True),
        scratch_types=[
            pltpu.VMEM((4 * N,), jnp.float32),
            pltpu.VMEM((2 * BUF,), jnp.float32),
            pltpu.SMEM((2,), jnp.int32),
            pltpu.SMEM((2,), jnp.float32),
            pltpu.SemaphoreType.DMA((2,)),
        ],
    )
    out_flat = k(x1d)
    return out_flat.reshape(8, 2, 8, K).transpose(0, 2, 1, 3).reshape(B, 2 * K)
